# baseline (speedup 1.0000x reference)
import functools

import jax
import jax.numpy as jnp
from jax import lax
from jax.experimental import pallas as pl
from jax.experimental.pallas import tpu as pltpu
from jax.experimental.pallas import tpu_sc as plsc


@functools.lru_cache(maxsize=None)
def _build(seq_len: int, nhid: int):
    info = plsc.get_sparse_core_info()
    nw = info.num_cores * info.num_subcores
    assert seq_len % nw == 0
    rows_per_w = seq_len // nw

    mesh = plsc.VectorSubcoreMesh(core_axis_name="c", subcore_axis_name="s")

    @functools.partial(
        pl.kernel,
        out_type=jax.ShapeDtypeStruct((seq_len, nhid), jnp.float32),
        mesh=mesh,
    )
    def pe_slice(pe_hbm, out_hbm):
        wid = lax.axis_index("s") * info.num_cores + lax.axis_index("c")
        base = wid * rows_per_w
        pltpu.sync_copy(
            pe_hbm.at[pl.ds(base, rows_per_w)],
            out_hbm.at[pl.ds(base, rows_per_w)],
        )

    return pe_slice


def kernel(x, pe):
    seq_len = x.shape[1]
    nhid = pe.shape[2]
    out2d = _build(seq_len, nhid)(pe.reshape(pe.shape[1], nhid))
    return out2d.reshape(1, seq_len, nhid)


# speedup vs baseline: 16.8933x; 16.8933x over previous
import functools

import jax
import jax.numpy as jnp
from jax import lax
from jax.experimental import pallas as pl
from jax.experimental.pallas import tpu as pltpu
from jax.experimental.pallas import tpu_sc as plsc


@functools.lru_cache(maxsize=None)
def _build(seq_len: int, nhid: int):
    info = plsc.get_sparse_core_info()
    nw = info.num_cores * info.num_subcores
    assert seq_len % nw == 0
    rows_per_w = seq_len // nw
    chunk = min(32, rows_per_w)
    nchunks = rows_per_w // chunk
    nbuf = min(2, nchunks)

    mesh = plsc.VectorSubcoreMesh(core_axis_name="c", subcore_axis_name="s")

    @functools.partial(
        pl.kernel,
        out_type=jax.ShapeDtypeStruct((seq_len, nhid), jnp.float32),
        mesh=mesh,
        scratch_types=[
            pltpu.VMEM((nbuf, chunk, nhid), jnp.float32),
            pltpu.SemaphoreType.DMA,
            pltpu.SemaphoreType.DMA,
        ],
    )
    def pe_slice(pe_hbm, out_hbm, buf, sem_in, sem_out):
        wid = lax.axis_index("s") * info.num_cores + lax.axis_index("c")
        base = wid * rows_per_w

        def src(i):
            return pe_hbm.at[pl.ds(base + i * chunk, chunk)]

        def dst(i):
            return out_hbm.at[pl.ds(base + i * chunk, chunk)]

        ins = [None] * nchunks
        outs = [None] * nchunks
        ins[0] = pltpu.async_copy(src(0), buf.at[0], sem_in)
        for i in range(nchunks):
            if i + 1 < nchunks:
                if i >= 1:
                    outs[i - 1].wait()
                ins[i + 1] = pltpu.async_copy(
                    src(i + 1), buf.at[(i + 1) % nbuf], sem_in)
            ins[i].wait()
            outs[i] = pltpu.async_copy(buf.at[i % nbuf], dst(i), sem_out)
        for i in range(max(0, nchunks - 2), nchunks):
            outs[i].wait()

    return pe_slice


def kernel(x, pe):
    seq_len = x.shape[1]
    nhid = pe.shape[2]
    out2d = _build(seq_len, nhid)(pe.reshape(pe.shape[1], nhid))
    return out2d.reshape(1, seq_len, nhid)


# speedup vs baseline: 39.1131x; 2.3153x over previous
import functools

import jax
import jax.numpy as jnp
from jax.experimental import pallas as pl
from jax.experimental.pallas import tpu as pltpu


@functools.lru_cache(maxsize=None)
def _build(seq_len: int, nhid: int):
    blk = 512
    assert seq_len % blk == 0
    grid = seq_len // blk

    def body(in_ref, out_ref):
        out_ref[...] = in_ref[...]

    return pl.pallas_call(
        body,
        grid=(grid,),
        in_specs=[pl.BlockSpec((blk, nhid), lambda i: (i, 0))],
        out_specs=pl.BlockSpec((blk, nhid), lambda i: (i, 0)),
        out_shape=jax.ShapeDtypeStruct((seq_len, nhid), jnp.float32),
    )


def kernel(x, pe):
    seq_len = x.shape[1]
    nhid = pe.shape[2]
    out2d = _build(seq_len, nhid)(pe.reshape(pe.shape[1], nhid))
    return out2d.reshape(1, seq_len, nhid)
